# baseline (speedup 1.0000x reference)
import functools

import jax
import jax.numpy as jnp
from jax import lax
from jax.experimental import pallas as pl
from jax.experimental.pallas import tpu as pltpu
from jax.experimental.pallas import tpu_sc as plsc

NC = 2
NS = 16
NW = NC * NS

RG = 16
NBUF = 2
L = 16


def _make_emb(BAT, HIST, V, D):
    assert BAT % NW == 0
    rpw = BAT // NW
    assert rpw % RG == 0 and rpw % 8 == 0
    ngrp = rpw // RG
    hp = (HIST + L - 1) // L * L

    mesh = plsc.VectorSubcoreMesh(core_axis_name="c", subcore_axis_name="s")

    @functools.partial(
        pl.kernel,
        mesh=mesh,
        out_type=jax.ShapeDtypeStruct((BAT, HIST, D), jnp.float32),
        scratch_types=[
            pltpu.VMEM((hp * rpw,), jnp.int32),
            pltpu.VMEM((rpw, HIST), jnp.int32),
            pltpu.VMEM((NBUF * RG, HIST, D), jnp.float32),
            pltpu.SemaphoreType.DMA((NBUF,)),
            pltpu.SemaphoreType.DMA((NBUF,)),
        ],
        compiler_params=pltpu.CompilerParams(use_tc_tiling_on_sc=False,
                                             needs_layout_passes=False),
    )
    def emb(xt_hbm, table_hbm, out_hbm, idxt_v, idx_v, rows_v, gsem, wsem):
        wid = lax.axis_index("s") * NC + lax.axis_index("c")
        base = wid * rpw
        for h in range(HIST):
            pltpu.async_copy(
                xt_hbm.at[h, pl.ds(base, rpw)],
                idxt_v.at[pl.ds(h * rpw, rpw)],
                gsem.at[0],
            )
        for h in range(HIST):
            pltpu.make_async_copy(
                xt_hbm.at[h, pl.ds(base, rpw)],
                idxt_v.at[pl.ds(h * rpw, rpw)],
                gsem.at[0],
            ).wait()

        lanes = lax.iota(jnp.int32, L)
        lanes_rpw = lanes * rpw

        def repack(r, _):
            r_vec = lanes * 0 + r
            for k in range(hp // L):
                pos = lanes_rpw + (k * L * rpw + r)
                v = plsc.load_gather(idxt_v, [pos])
                c_vec = lanes + (k * L)
                if (k + 1) * L <= HIST:
                    plsc.store_scatter(idx_v, [r_vec, c_vec], v)
                else:
                    plsc.store_scatter(idx_v, [r_vec, c_vec], v,
                                       mask=lanes < (HIST - k * L))
            return ()

        lax.fori_loop(0, rpw, repack, ())

        def fire(g, buf):
            for k in range(RG):
                pltpu.async_copy(
                    table_hbm.at[idx_v.at[g * RG + k]],
                    rows_v.at[buf * RG + k],
                    gsem.at[buf],
                )

        fire(0, 0)

        def body(g, _):
            cur = g % 2
            nxt = 1 - cur

            @pl.when(g + 1 < ngrp)
            def _():
                @pl.when(g > 0)
                def _():
                    pltpu.make_async_copy(
                        rows_v.at[pl.ds(nxt * RG, RG)],
                        out_hbm.at[pl.ds(base, RG)],
                        wsem.at[nxt],
                    ).wait()
                fire(g + 1, nxt)

            pltpu.make_async_copy(
                out_hbm.at[pl.ds(0, RG)],
                rows_v.at[pl.ds(cur * RG, RG)],
                gsem.at[cur],
            ).wait()
            pltpu.async_copy(
                rows_v.at[pl.ds(cur * RG, RG)],
                out_hbm.at[pl.ds(base + g * RG, RG)],
                wsem.at[cur],
            )
            return ()

        lax.fori_loop(0, ngrp, body, ())

        for buf in range(NBUF):
            pltpu.make_async_copy(
                rows_v.at[pl.ds(buf * RG, RG)],
                out_hbm.at[pl.ds(base, RG)],
                wsem.at[buf],
            ).wait()

    return emb


def kernel(x, embedding_matrix):
    bat, hist = x.shape
    V, D = embedding_matrix.shape
    return _make_emb(bat, hist, V, D)(x.T.astype(jnp.int32), embedding_matrix)


# speedup vs baseline: 1.0201x; 1.0201x over previous
import functools

import jax
import jax.numpy as jnp
from jax import lax
from jax.experimental import pallas as pl
from jax.experimental.pallas import tpu as pltpu
from jax.experimental.pallas import tpu_sc as plsc

NC = 2
NS = 16
NW = NC * NS

RG = 32
NBUF = 2


def _make_emb(BAT, HIST, V, D):
    assert BAT % NW == 0
    rpw = BAT // NW
    assert rpw % RG == 0
    ngrp = rpw // RG

    mesh = plsc.VectorSubcoreMesh(core_axis_name="c", subcore_axis_name="s")

    @functools.partial(
        pl.kernel,
        mesh=mesh,
        out_type=jax.ShapeDtypeStruct((BAT, HIST, D), jnp.float32),
        scratch_types=[
            pltpu.VMEM((rpw, HIST), jnp.int32),
            pltpu.VMEM((NBUF * RG, HIST, D), jnp.float32),
            pltpu.SemaphoreType.DMA((NBUF,)),
            pltpu.SemaphoreType.DMA((NBUF,)),
        ],
        compiler_params=pltpu.CompilerParams(use_tc_tiling_on_sc=False),
    )
    def emb(x_hbm, table_hbm, out_hbm, idx_v, rows_v, gsem, wsem):
        wid = lax.axis_index("s") * NC + lax.axis_index("c")
        base = wid * rpw
        pltpu.sync_copy(x_hbm.at[pl.ds(base, rpw)], idx_v)

        def fire(g, buf):
            for k in range(RG):
                pltpu.async_copy(
                    table_hbm.at[idx_v.at[g * RG + k]],
                    rows_v.at[buf * RG + k],
                    gsem.at[buf],
                )

        fire(0, 0)

        def body(g, _):
            cur = g % 2
            nxt = 1 - cur

            @pl.when(g + 1 < ngrp)
            def _():
                @pl.when(g > 0)
                def _():
                    pltpu.make_async_copy(
                        rows_v.at[pl.ds(nxt * RG, RG)],
                        out_hbm.at[pl.ds(base, RG)],
                        wsem.at[nxt],
                    ).wait()
                fire(g + 1, nxt)

            pltpu.make_async_copy(
                out_hbm.at[pl.ds(0, RG)],
                rows_v.at[pl.ds(cur * RG, RG)],
                gsem.at[cur],
            ).wait()
            pltpu.async_copy(
                rows_v.at[pl.ds(cur * RG, RG)],
                out_hbm.at[pl.ds(base + g * RG, RG)],
                wsem.at[cur],
            )
            return ()

        lax.fori_loop(0, ngrp, body, ())

        for buf in range(NBUF):
            pltpu.make_async_copy(
                rows_v.at[pl.ds(buf * RG, RG)],
                out_hbm.at[pl.ds(base, RG)],
                wsem.at[buf],
            ).wait()

    return emb


def kernel(x, embedding_matrix):
    bat, hist = x.shape
    V, D = embedding_matrix.shape
    return _make_emb(bat, hist, V, D)(x.astype(jnp.int32), embedding_matrix)
